# Initial kernel scaffold; baseline (speedup 1.0000x reference)
import functools

import jax
import jax.numpy as jnp
from jax.experimental import pallas as pl
from jax.experimental.pallas import tpu as pltpu
from jax.experimental.pallas import tpu_sc as plsc

N = 10000
K = 32
D = 128
H = 8
DH = D // H

_B = 400
_GW = 100


def _sc_gather(x_bf, idx):
    ni = idx.shape[1]
    d = x_bf.shape[1]
    mesh = plsc.VectorSubcoreMesh(core_axis_name="core", subcore_axis_name="subcore")

    @functools.partial(
        pl.kernel,
        out_type=jax.ShapeDtypeStruct((ni, d), x_bf.dtype),
        mesh=mesh,
    )
    def gather_kernel(x_hbm, i_hbm, o_hbm):
        def body(i_vmem, o_vmem):
            pltpu.sync_copy(x_hbm.at[i_vmem.at[0]], o_vmem)

        pltpu.emit_pipeline(
            body,
            grid=(ni // _GW,),
            in_specs=[pl.BlockSpec((1, _GW), index_map=lambda i: (0, i))],
            out_specs=[pl.BlockSpec((_GW, d), index_map=lambda i: (i, 0))],
            core_axis_name=("core", "subcore"),
            dimension_semantics=(pltpu.PARALLEL,),
        )(i_hbm, o_hbm)

    return gather_kernel(x_bf, idx)


def _ln(y, g, b):
    mu = jnp.mean(y, axis=-1, keepdims=True)
    var = jnp.mean((y - mu) * (y - mu), axis=-1, keepdims=True)
    return (y - mu) / jnp.sqrt(var + 1e-5) * g + b


def _dot(a, b):
    return jax.lax.dot_general(
        a, b, (((1,), (0,)), ((), ())), preferred_element_type=jnp.float32
    )


def _tc_body(x_ref, g_ref, wqT, wkT, wvT, bq, bk, bv, msel, mselT, owT, ob,
             w1T, fb1, w2T, fb2, g1r, b1r, g2r, b2r, o_ref):
    xb = x_ref[...]
    xbf = xb.astype(jnp.bfloat16)
    gb = g_ref[...]

    q = _dot(xbf, wqT[...]) + bq[...]
    kk = _dot(gb, wkT[...]) + bk[...]
    vv = _dot(gb, wvT[...]) + bv[...]

    qk = (kk.reshape(_B, K, D) * q[:, None, :]).reshape(_B * K, D)
    s = _dot(qk.astype(jnp.bfloat16), msel[...]) * (1.0 / (DH ** 0.5))
    s3 = s.reshape(_B, K, H)
    m = jnp.max(s3, axis=1, keepdims=True)
    e = jnp.exp(s3 - m)
    a = e / jnp.sum(e, axis=1, keepdims=True)
    ab = _dot(a.reshape(_B * K, H).astype(jnp.bfloat16), mselT[...])
    o = jnp.sum(ab.reshape(_B, K, D) * vv.reshape(_B, K, D), axis=1)

    o = _dot(o.astype(jnp.bfloat16), owT[...]) + ob[...]
    x1 = _ln(xb + o, g1r[...], b1r[...])
    h1 = jnp.maximum(_dot(x1.astype(jnp.bfloat16), w1T[...]) + fb1[...], 0.0)
    f = _dot(h1.astype(jnp.bfloat16), w2T[...]) + fb2[...]
    o_ref[...] = _ln(x1 + f, g2r[...], b2r[...])


def kernel(x, attention_samples, in_proj_w, in_proj_b, out_w, out_b,
           ffn_w1, ffn_b1, ffn_w2, ffn_b2, g1, b1, g2, b2):
    bf = jnp.bfloat16
    x_bf = x.astype(bf)
    idx = attention_samples.astype(jnp.int32).reshape(1, N * K)
    gathered = _sc_gather(x_bf, idx)

    wqT = in_proj_w[:D].T.astype(bf)
    wkT = in_proj_w[D:2 * D].T.astype(bf)
    wvT = in_proj_w[2 * D:].T.astype(bf)
    bq = in_proj_b[:D].reshape(1, D)
    bk = in_proj_b[D:2 * D].reshape(1, D)
    bv = in_proj_b[2 * D:].reshape(1, D)
    msel = (jnp.arange(D)[:, None] // DH == jnp.arange(H)[None, :]).astype(bf)
    mselT = msel.T
    owT = out_w.T.astype(bf)
    w1T = ffn_w1.T.astype(bf)
    w2T = ffn_w2.T.astype(bf)

    full = lambda shape: pl.BlockSpec(shape, lambda i: (0, 0))
    out = pl.pallas_call(
        _tc_body,
        grid=(N // _B,),
        in_specs=[
            pl.BlockSpec((_B, D), lambda i: (i, 0)),
            pl.BlockSpec((_B * K, D), lambda i: (i, 0)),
            full((D, D)), full((D, D)), full((D, D)),
            full((1, D)), full((1, D)), full((1, D)),
            full((D, H)), full((H, D)),
            full((D, D)), full((1, D)),
            full((D, 4 * D)), full((1, 4 * D)),
            full((4 * D, D)), full((1, D)),
            full((1, D)), full((1, D)), full((1, D)), full((1, D)),
        ],
        out_specs=pl.BlockSpec((_B, D), lambda i: (i, 0)),
        out_shape=jax.ShapeDtypeStruct((N, D), jnp.float32),
    )(
        x, gathered, wqT, wkT, wvT, bq, bk, bv, msel, mselT,
        owT, out_b.reshape(1, D), w1T, ffn_b1.reshape(1, 4 * D),
        w2T, ffn_b2.reshape(1, D),
        g1.reshape(1, D), b1.reshape(1, D), g2.reshape(1, D), b2.reshape(1, D),
    )
    return out


# speedup vs baseline: 3.9405x; 3.9405x over previous
import functools

import jax
import jax.numpy as jnp
from jax.experimental import pallas as pl
from jax.experimental.pallas import tpu as pltpu
from jax.experimental.pallas import tpu_sc as plsc

N = 10000
K = 32
D = 128
H = 8
DH = D // H

_B = 400
_GW = 128


def _sc_gather(x_bf, idx):
    ni = idx.shape[1]
    d = x_bf.shape[1]
    mesh = plsc.VectorSubcoreMesh(core_axis_name="core", subcore_axis_name="subcore")

    @functools.partial(
        pl.kernel,
        out_type=jax.ShapeDtypeStruct((ni, d), x_bf.dtype),
        mesh=mesh,
    )
    def gather_kernel(x_hbm, i_hbm, o_hbm):
        def body(i_vmem, o_vmem):
            pltpu.sync_copy(x_hbm.at[i_vmem.at[0]], o_vmem)

        pltpu.emit_pipeline(
            body,
            grid=(ni // _GW,),
            in_specs=[pl.BlockSpec((1, _GW), index_map=lambda i: (0, i))],
            out_specs=[pl.BlockSpec((_GW, d), index_map=lambda i: (i, 0))],
            core_axis_name=("core", "subcore"),
            dimension_semantics=(pltpu.PARALLEL,),
        )(i_hbm, o_hbm)

    return gather_kernel(x_bf, idx)


def _ln(y, g, b):
    mu = jnp.mean(y, axis=-1, keepdims=True)
    var = jnp.mean((y - mu) * (y - mu), axis=-1, keepdims=True)
    return (y - mu) / jnp.sqrt(var + 1e-5) * g + b


def _dot(a, b):
    return jax.lax.dot_general(
        a, b, (((1,), (0,)), ((), ())), preferred_element_type=jnp.float32
    )


def _tc_body(x_ref, g_ref, wqT, wkT, wvT, bq, bk, bv, msel, mselT, owT, ob,
             w1T, fb1, w2T, fb2, g1r, b1r, g2r, b2r, o_ref):
    xb = x_ref[...]
    xbf = xb.astype(jnp.bfloat16)
    gb = g_ref[...].astype(jnp.bfloat16)

    q = _dot(xbf, wqT[...]) + bq[...]
    kk = _dot(gb, wkT[...]) + bk[...]
    vv = _dot(gb, wvT[...]) + bv[...]

    qk = (kk.reshape(_B, K, D) * q[:, None, :]).reshape(_B * K, D)
    s = _dot(qk.astype(jnp.bfloat16), msel[...]) * (1.0 / (DH ** 0.5))
    s3 = s.reshape(_B, K, H)
    m = jnp.max(s3, axis=1, keepdims=True)
    e = jnp.exp(s3 - m)
    a = e / jnp.sum(e, axis=1, keepdims=True)
    ab = _dot(a.reshape(_B * K, H).astype(jnp.bfloat16), mselT[...])
    o = jnp.sum(ab.reshape(_B, K, D) * vv.reshape(_B, K, D), axis=1)

    o = _dot(o.astype(jnp.bfloat16), owT[...]) + ob[...]
    x1 = _ln(xb + o, g1r[...], b1r[...])
    h1 = jnp.maximum(_dot(x1.astype(jnp.bfloat16), w1T[...]) + fb1[...], 0.0)
    f = _dot(h1.astype(jnp.bfloat16), w2T[...]) + fb2[...]
    o_ref[...] = _ln(x1 + f, g2r[...], b2r[...])


def kernel(x, attention_samples, in_proj_w, in_proj_b, out_w, out_b,
           ffn_w1, ffn_b1, ffn_w2, ffn_b2, g1, b1, g2, b2):
    bf = jnp.bfloat16
    idx = attention_samples.astype(jnp.int32).reshape(1, N * K)
    gathered = _sc_gather(x, idx)

    wqT = in_proj_w[:D].T.astype(bf)
    wkT = in_proj_w[D:2 * D].T.astype(bf)
    wvT = in_proj_w[2 * D:].T.astype(bf)
    bq = in_proj_b[:D].reshape(1, D)
    bk = in_proj_b[D:2 * D].reshape(1, D)
    bv = in_proj_b[2 * D:].reshape(1, D)
    msel = (jnp.arange(D)[:, None] // DH == jnp.arange(H)[None, :]).astype(bf)
    mselT = msel.T
    owT = out_w.T.astype(bf)
    w1T = ffn_w1.T.astype(bf)
    w2T = ffn_w2.T.astype(bf)

    full = lambda shape: pl.BlockSpec(shape, lambda i: (0, 0))
    out = pl.pallas_call(
        _tc_body,
        grid=(N // _B,),
        in_specs=[
            pl.BlockSpec((_B, D), lambda i: (i, 0)),
            pl.BlockSpec((_B * K, D), lambda i: (i, 0)),
            full((D, D)), full((D, D)), full((D, D)),
            full((1, D)), full((1, D)), full((1, D)),
            full((D, H)), full((H, D)),
            full((D, D)), full((1, D)),
            full((D, 4 * D)), full((1, 4 * D)),
            full((4 * D, D)), full((1, D)),
            full((1, D)), full((1, D)), full((1, D)), full((1, D)),
        ],
        out_specs=pl.BlockSpec((_B, D), lambda i: (i, 0)),
        out_shape=jax.ShapeDtypeStruct((N, D), jnp.float32),
    )(
        x, gathered, wqT, wkT, wvT, bq, bk, bv, msel, mselT,
        owT, out_b.reshape(1, D), w1T, ffn_b1.reshape(1, 4 * D),
        w2T, ffn_b2.reshape(1, D),
        g1.reshape(1, D), b1.reshape(1, D), g2.reshape(1, D), b2.reshape(1, D),
    )
    return out


# speedup vs baseline: 4.5357x; 1.1511x over previous
import functools

import jax
import jax.numpy as jnp
from jax.experimental import pallas as pl
from jax.experimental.pallas import tpu as pltpu
from jax.experimental.pallas import tpu_sc as plsc

N = 10000
K = 32
D = 128
H = 8
DH = D // H

_B = 400
_GW = 128


def _sc_gather(table, idx):
    ni = idx.shape[1]
    d = table.shape[1]
    mesh = plsc.VectorSubcoreMesh(core_axis_name="core", subcore_axis_name="subcore")

    @functools.partial(
        pl.kernel,
        out_type=jax.ShapeDtypeStruct((ni, d), table.dtype),
        mesh=mesh,
    )
    def gather_kernel(x_hbm, i_hbm, o_hbm):
        def body(i_vmem, o_vmem):
            pltpu.sync_copy(x_hbm.at[i_vmem.at[0]], o_vmem)

        pltpu.emit_pipeline(
            body,
            grid=(ni // _GW,),
            in_specs=[pl.BlockSpec((1, _GW), index_map=lambda i: (0, i))],
            out_specs=[pl.BlockSpec((_GW, d), index_map=lambda i: (i, 0))],
            core_axis_name=("core", "subcore"),
            dimension_semantics=(pltpu.PARALLEL,),
        )(i_hbm, o_hbm)

    return gather_kernel(table, idx)


def _ln(y, g, b):
    mu = jnp.mean(y, axis=-1, keepdims=True)
    var = jnp.mean((y - mu) * (y - mu), axis=-1, keepdims=True)
    return (y - mu) / jnp.sqrt(var + 1e-5) * g + b


def _dot(a, b):
    return jax.lax.dot_general(
        a, b, (((1,), (0,)), ((), ())), preferred_element_type=jnp.float32
    )


def _kv_pack_body(x_ref, wkvT_ref, o_ref):
    xbf = x_ref[...].astype(jnp.bfloat16)
    kv = _dot(xbf, wkvT_ref[...])
    p = kv[:, :D].astype(jnp.bfloat16).astype(jnp.float32)
    q = kv[:, D:].astype(jnp.bfloat16).astype(jnp.float32)
    pu = jax.lax.bitcast_convert_type(p, jnp.uint32) >> 16
    qu = jax.lax.bitcast_convert_type(q, jnp.uint32) & jnp.uint32(0xFFFF0000)
    o_ref[...] = jax.lax.bitcast_convert_type(pu | qu, jnp.int32)


def _tc_body(x_ref, g_ref, wqT, bq, msel, mselT, bvr, owT, ob,
             w1T, fb1, w2T, fb2, g1r, b1r, g2r, b2r, o_ref):
    bf = jnp.bfloat16
    xb = x_ref[...]
    xbf = xb.astype(bf)
    gi = g_ref[...]
    kk = jax.lax.bitcast_convert_type(gi << 16, jnp.float32)
    vv = jax.lax.bitcast_convert_type(gi & jnp.int32(-65536), jnp.float32)

    q = _dot(xbf, wqT[...]) + bq[...]
    qk = (kk.reshape(_B, K, D) * q[:, None, :]).reshape(_B * K, D)
    s = _dot(qk.astype(bf), msel[...])
    e = jnp.exp(s.reshape(_B, K, H))
    s8 = jnp.sum(e, axis=1)
    eb = _dot(e.reshape(_B * K, H).astype(bf), mselT[...])
    u = jnp.sum(eb.reshape(_B, K, D) * vv.reshape(_B, K, D), axis=1)
    sb = _dot(s8.astype(bf), mselT[...])
    o = u / sb + bvr[...]

    o = _dot(o.astype(bf), owT[...]) + ob[...]
    x1 = _ln(xb + o, g1r[...], b1r[...])
    h1 = jnp.maximum(_dot(x1.astype(bf), w1T[...]) + fb1[...], 0.0)
    f = _dot(h1.astype(bf), w2T[...]) + fb2[...]
    o_ref[...] = _ln(x1 + f, g2r[...], b2r[...])


def kernel(x, attention_samples, in_proj_w, in_proj_b, out_w, out_b,
           ffn_w1, ffn_b1, ffn_w2, ffn_b2, g1, b1, g2, b2):
    bf = jnp.bfloat16
    idx = attention_samples.astype(jnp.int32).reshape(1, N * K)

    wkvT = in_proj_w[D:].T.astype(bf)
    kv_pk = pl.pallas_call(
        _kv_pack_body,
        grid=(1,),
        in_specs=[pl.BlockSpec((N, D), lambda i: (0, 0)),
                  pl.BlockSpec((D, 2 * D), lambda i: (0, 0))],
        out_specs=pl.BlockSpec((N, D), lambda i: (0, 0)),
        out_shape=jax.ShapeDtypeStruct((N, D), jnp.int32),
    )(x, wkvT)

    gathered = _sc_gather(kv_pk, idx)

    scale = 1.0 / (DH ** 0.5)
    wqT = (in_proj_w[:D].T * scale).astype(bf)
    bq = in_proj_b[:D].reshape(1, D) * scale
    bvr = in_proj_b[2 * D:].reshape(1, D)
    msel = (jnp.arange(D)[:, None] // DH == jnp.arange(H)[None, :]).astype(bf)
    mselT = msel.T
    owT = out_w.T.astype(bf)
    w1T = ffn_w1.T.astype(bf)
    w2T = ffn_w2.T.astype(bf)

    full = lambda shape: pl.BlockSpec(shape, lambda i: (0, 0))
    out = pl.pallas_call(
        _tc_body,
        grid=(N // _B,),
        in_specs=[
            pl.BlockSpec((_B, D), lambda i: (i, 0)),
            pl.BlockSpec((_B * K, D), lambda i: (i, 0)),
            full((D, D)), full((1, D)),
            full((D, H)), full((H, D)), full((1, D)),
            full((D, D)), full((1, D)),
            full((D, 4 * D)), full((1, 4 * D)),
            full((4 * D, D)), full((1, D)),
            full((1, D)), full((1, D)), full((1, D)), full((1, D)),
        ],
        out_specs=pl.BlockSpec((_B, D), lambda i: (i, 0)),
        out_shape=jax.ShapeDtypeStruct((N, D), jnp.float32),
    )(
        x, gathered, wqT, bq, msel, mselT, bvr,
        owT, out_b.reshape(1, D), w1T, ffn_b1.reshape(1, 4 * D),
        w2T, ffn_b2.reshape(1, D),
        g1.reshape(1, D), b1.reshape(1, D), g2.reshape(1, D), b2.reshape(1, D),
    )
    return out


# speedup vs baseline: 5.2096x; 1.1486x over previous
import functools

import jax
import jax.numpy as jnp
from jax.experimental import pallas as pl
from jax.experimental.pallas import tpu as pltpu
from jax.experimental.pallas import tpu_sc as plsc

N = 10000
K = 32
D = 128
H = 8
DH = D // H

_B = 400
_GW = 128
_NCH = 5
_CN = N // _NCH


def _sc_gather(table, idx):
    ni = idx.shape[1]
    d = table.shape[1]
    mesh = plsc.VectorSubcoreMesh(core_axis_name="core", subcore_axis_name="subcore")

    @functools.partial(
        pl.kernel,
        out_type=jax.ShapeDtypeStruct((ni, d), table.dtype),
        mesh=mesh,
    )
    def gather_kernel(x_hbm, i_hbm, o_hbm):
        def body(i_vmem, o_vmem):
            pltpu.sync_copy(x_hbm.at[i_vmem.at[0]], o_vmem)

        pltpu.emit_pipeline(
            body,
            grid=(ni // _GW,),
            in_specs=[pl.BlockSpec((1, _GW), index_map=lambda i: (0, i))],
            out_specs=[pl.BlockSpec((_GW, d), index_map=lambda i: (i, 0))],
            core_axis_name=("core", "subcore"),
            dimension_semantics=(pltpu.PARALLEL,),
        )(i_hbm, o_hbm)

    return gather_kernel(table, idx)


def _ln(y, g, b):
    mu = jnp.mean(y, axis=-1, keepdims=True)
    var = jnp.mean((y - mu) * (y - mu), axis=-1, keepdims=True)
    return (y - mu) / jnp.sqrt(var + 1e-5) * g + b


def _dot(a, b):
    return jax.lax.dot_general(
        a, b, (((1,), (0,)), ((), ())), preferred_element_type=jnp.float32
    )


def _kv_pack_body(x_ref, wkvT_ref, o_ref):
    xbf = x_ref[...].astype(jnp.bfloat16)
    kv = _dot(xbf, wkvT_ref[...])
    p = kv[:, :D].astype(jnp.bfloat16).astype(jnp.float32)
    q = kv[:, D:].astype(jnp.bfloat16).astype(jnp.float32)
    pu = jax.lax.bitcast_convert_type(p, jnp.uint32) >> 16
    qu = jax.lax.bitcast_convert_type(q, jnp.uint32) & jnp.uint32(0xFFFF0000)
    o_ref[...] = jax.lax.bitcast_convert_type(pu | qu, jnp.int32)


def _tc_body(x_ref, g_ref, wqT, bq, msel, mselT, bvr, owT, ob,
             w1T, fb1, w2T, fb2, g1r, b1r, g2r, b2r, o_ref):
    bf = jnp.bfloat16
    xb = x_ref[...]
    xbf = xb.astype(bf)
    gi = g_ref[...]
    kk = jax.lax.bitcast_convert_type(gi << 16, jnp.float32)
    vv = jax.lax.bitcast_convert_type(gi & jnp.int32(-65536), jnp.float32)

    q = _dot(xbf, wqT[...]) + bq[...]
    qk = (kk.reshape(_B, K, D) * q[:, None, :]).reshape(_B * K, D)
    s = _dot(qk.astype(bf), msel[...])
    e = jnp.exp(s.reshape(_B, K, H))
    s8 = jnp.sum(e, axis=1)
    eb = _dot(e.reshape(_B * K, H).astype(bf), mselT[...])
    u = jnp.sum(eb.reshape(_B, K, D) * vv.reshape(_B, K, D), axis=1)
    sb = _dot(s8.astype(bf), mselT[...])
    o = u / sb + bvr[...]

    o = _dot(o.astype(bf), owT[...]) + ob[...]
    x1 = _ln(xb + o, g1r[...], b1r[...])
    h1 = jnp.maximum(_dot(x1.astype(bf), w1T[...]) + fb1[...], 0.0)
    f = _dot(h1.astype(bf), w2T[...]) + fb2[...]
    o_ref[...] = _ln(x1 + f, g2r[...], b2r[...])


def kernel(x, attention_samples, in_proj_w, in_proj_b, out_w, out_b,
           ffn_w1, ffn_b1, ffn_w2, ffn_b2, g1, b1, g2, b2):
    bf = jnp.bfloat16
    idx = attention_samples.astype(jnp.int32).reshape(1, N * K)

    wkvT = in_proj_w[D:].T.astype(bf)
    kv_pk = pl.pallas_call(
        _kv_pack_body,
        grid=(1,),
        in_specs=[pl.BlockSpec((N, D), lambda i: (0, 0)),
                  pl.BlockSpec((D, 2 * D), lambda i: (0, 0))],
        out_specs=pl.BlockSpec((N, D), lambda i: (0, 0)),
        out_shape=jax.ShapeDtypeStruct((N, D), jnp.int32),
    )(x, wkvT)

    scale = 1.0 / (DH ** 0.5)
    wqT = (in_proj_w[:D].T * scale).astype(bf)
    bq = in_proj_b[:D].reshape(1, D) * scale
    bvr = in_proj_b[2 * D:].reshape(1, D)
    msel = (jnp.arange(D)[:, None] // DH == jnp.arange(H)[None, :]).astype(bf)
    mselT = msel.T
    owT = out_w.T.astype(bf)
    w1T = ffn_w1.T.astype(bf)
    w2T = ffn_w2.T.astype(bf)

    full = lambda shape: pl.BlockSpec(shape, lambda i: (0, 0))
    tc_call = pl.pallas_call(
        _tc_body,
        grid=(_CN // _B,),
        in_specs=[
            pl.BlockSpec((_B, D), lambda i: (i, 0)),
            pl.BlockSpec((_B * K, D), lambda i: (i, 0)),
            full((D, D)), full((1, D)),
            full((D, H)), full((H, D)), full((1, D)),
            full((D, D)), full((1, D)),
            full((D, 4 * D)), full((1, 4 * D)),
            full((4 * D, D)), full((1, D)),
            full((1, D)), full((1, D)), full((1, D)), full((1, D)),
        ],
        out_specs=pl.BlockSpec((_B, D), lambda i: (i, 0)),
        out_shape=jax.ShapeDtypeStruct((_CN, D), jnp.float32),
    )
    consts = (
        wqT, bq, msel, mselT, bvr,
        owT, out_b.reshape(1, D), w1T, ffn_b1.reshape(1, 4 * D),
        w2T, ffn_b2.reshape(1, D),
        g1.reshape(1, D), b1.reshape(1, D), g2.reshape(1, D), b2.reshape(1, D),
    )
    outs = []
    for c in range(_NCH):
        g_c = _sc_gather(kv_pk, idx[:, c * _CN * K:(c + 1) * _CN * K])
        outs.append(tc_call(x[c * _CN:(c + 1) * _CN], g_c, *consts))
    return jnp.concatenate(outs, axis=0)
